# baseline (speedup 1.0000x reference)
import jax
import jax.numpy as jnp
from jax.experimental import pallas as pl

_NUM_ACTIONS = 1000


def _onehot_kernel(enc_ref, out_ref):
    enc = enc_ref[...]
    bm, t = enc.shape
    pos = jax.lax.broadcasted_iota(jnp.int32, (bm, t), 1)
    key = jnp.where(enc >= 0, (pos + 1) * 1024 + enc, 0)
    m = jnp.max(key, axis=1)
    action = jnp.where(m > 0, jnp.bitwise_and(m, 1023), -1)
    aidx = jax.lax.broadcasted_iota(jnp.int32, (bm, _NUM_ACTIONS), 1)
    out_ref[...] = (aidx == action[:, None]).astype(jnp.int32)


def kernel(encodings):
    m, t = encodings.shape
    bm = 512
    return pl.pallas_call(
        _onehot_kernel,
        grid=(m // bm,),
        in_specs=[pl.BlockSpec((bm, t), lambda i: (i, 0))],
        out_specs=pl.BlockSpec((bm, _NUM_ACTIONS), lambda i: (i, 0)),
        out_shape=jax.ShapeDtypeStruct((m, _NUM_ACTIONS), jnp.int32),
    )(encodings)


# speedup vs baseline: 1.1038x; 1.1038x over previous
import jax
import jax.numpy as jnp
from jax.experimental import pallas as pl

_NUM_ACTIONS = 1000


def _onehot_kernel(enc_ref, out_ref):
    enc = enc_ref[...]
    bm, t = enc.shape
    pos = jax.lax.broadcasted_iota(jnp.int32, (bm, t), 1)
    key = jnp.where(enc >= 0, (pos + 1) * 1024 + enc, 0)
    m = jnp.max(key, axis=1)
    action = jnp.where(m > 0, jnp.bitwise_and(m, 1023), -1)
    aidx = jax.lax.broadcasted_iota(jnp.int32, (bm, _NUM_ACTIONS), 1)
    out_ref[...] = (aidx == action[:, None]).astype(jnp.int32)


def kernel(encodings):
    m, t = encodings.shape
    bm = 2048
    return pl.pallas_call(
        _onehot_kernel,
        grid=(m // bm,),
        in_specs=[pl.BlockSpec((bm, t), lambda i: (i, 0))],
        out_specs=pl.BlockSpec((bm, _NUM_ACTIONS), lambda i: (i, 0)),
        out_shape=jax.ShapeDtypeStruct((m, _NUM_ACTIONS), jnp.int32),
    )(encodings)


# speedup vs baseline: 4.4555x; 4.0364x over previous
import jax
import jax.numpy as jnp
from jax.experimental import pallas as pl

_NUM_ACTIONS = 1000


def _onehot_kernel(enc_ref, out_ref):
    enc = enc_ref[...]
    t, bm = enc.shape
    pos = jax.lax.broadcasted_iota(jnp.int32, (t, bm), 0)
    key = jnp.where(enc >= 0, (pos + 1) * 1024 + enc, 0)
    m = jnp.max(key, axis=0, keepdims=True)
    action = jnp.where(m > 0, jnp.bitwise_and(m, 1023), -1)
    aidx = jax.lax.broadcasted_iota(jnp.int32, (_NUM_ACTIONS, bm), 0)
    out_ref[...] = (aidx == action).astype(jnp.int32)


def kernel(encodings):
    m, t = encodings.shape
    bm = 2048
    enc_t = encodings.T
    out_t = pl.pallas_call(
        _onehot_kernel,
        grid=(m // bm,),
        in_specs=[pl.BlockSpec((t, bm), lambda i: (0, i))],
        out_specs=pl.BlockSpec((_NUM_ACTIONS, bm), lambda i: (0, i)),
        out_shape=jax.ShapeDtypeStruct((_NUM_ACTIONS, m), jnp.int32),
    )(enc_t)
    return out_t.T


# speedup vs baseline: 4.4802x; 1.0055x over previous
import jax
import jax.numpy as jnp
from jax.experimental import pallas as pl

_NUM_ACTIONS = 1000


def _onehot_kernel(enc_ref, out_ref):
    enc = enc_ref[...]
    t, bm = enc.shape
    pos = jax.lax.broadcasted_iota(jnp.int32, (t, bm), 0)
    key = jnp.where(enc >= 0, (pos + 1) * 1024 + enc, 0)
    m = jnp.max(key, axis=0, keepdims=True)
    action = jnp.where(m > 0, jnp.bitwise_and(m, 1023), -1)
    aidx = jax.lax.broadcasted_iota(jnp.int32, (_NUM_ACTIONS, bm), 0)
    out_ref[...] = (aidx == action).astype(jnp.int32)


def kernel(encodings):
    m, t = encodings.shape
    bm = 4096
    enc_t = encodings.T
    out_t = pl.pallas_call(
        _onehot_kernel,
        grid=(m // bm,),
        in_specs=[pl.BlockSpec((t, bm), lambda i: (0, i))],
        out_specs=pl.BlockSpec((_NUM_ACTIONS, bm), lambda i: (0, i)),
        out_shape=jax.ShapeDtypeStruct((_NUM_ACTIONS, m), jnp.int32),
    )(enc_t)
    return out_t.T
